# Initial kernel scaffold; baseline (speedup 1.0000x reference)
import jax
import jax.numpy as jnp
from jax.experimental import pallas as pl


def kernel(x, edge_index, known_feature_mask):
    raise NotImplementedError("write your pallas kernel here")


# speedup vs baseline: 4.7838x; 4.7838x over previous
import functools

import jax
import jax.numpy as jnp
from jax import lax
from jax.experimental import pallas as pl
from jax.experimental.pallas import tpu as pltpu
from jax.experimental.pallas import tpu_sc as plsc

N = 10000
E = 320000
D = 128
K = 5000
ITERS = 10

NC = 2
NS = 16
NW = NC * NS

NPAD = 10240
HALF = NPAD // NC
ROWS_PT = HALF // NS
NREAL_HALF = 5000
ZPAD_W = HALF - NREAL_HALF
ZPAD0 = NREAL_HALF
ZPAD1 = HALF + NREAL_HALF

ECH = 128
ECHUNKS = 82
EPT = ECH * ECHUNKS
ECAP = EPT * NS

DCHUNKS = 157
EPAD_DEG = NS * DCHUNKS * ECH

KCH = 64
KCHUNKS = 3
KPT = KCH * KCHUNKS
KCAP = KPT * NS

f32 = jnp.float32
i32 = jnp.int32


@functools.lru_cache(maxsize=None)
def _mesh():
    return plsc.VectorSubcoreMesh(core_axis_name="c", subcore_axis_name="s",
                                  num_cores=NC, num_subcores=NS)


def _wid():
    return lax.axis_index("c") * NS + lax.axis_index("s")


def _zero_vmem(ref, rows):
    @pl.loop(0, rows)
    def _(r):
        for j in range(D // 16):
            ref[r, pl.ds(j * 16, 16)] = jnp.zeros((16,), f32)


def _rsqrt16(d):
    dd = jnp.maximum(d, 1.0)
    y = 1.0 / dd
    for _ in range(26):
        y = y * (1.5 - 0.5 * dd * y * y)
    return jnp.where(d > 0.0, y, 0.0)


@functools.lru_cache(maxsize=None)
def _make_setup1():
    @functools.partial(
        pl.kernel,
        out_type=jax.ShapeDtypeStruct((NPAD,), f32),
        mesh=_mesh(),
        scratch_types=[
            pltpu.VMEM_SHARED((NPAD,), f32),
            pltpu.VMEM((DCHUNKS, ECH), i32),
            pltpu.VMEM((ECH,), f32),
            pltpu.VMEM((NPAD // NS,), f32),
            pltpu.VMEM((ROWS_PT,), f32),
            pltpu.VMEM((ROWS_PT,), f32),
        ],
    )
    def setup1(erow_deg, disH, degS, didx, ones, zslice, degL, disL):
        c = lax.axis_index("c")
        s = lax.axis_index("s")

        @pl.loop(0, NPAD // NS // 16)
        def _(j):
            zslice[pl.ds(j * 16, 16)] = jnp.zeros((16,), f32)

        pltpu.sync_copy(zslice, degS.at[pl.ds(s * (NPAD // NS), NPAD // NS)])

        @pl.loop(0, ECH // 16)
        def _(j):
            ones[pl.ds(j * 16, 16)] = jnp.full((16,), 1.0, f32)

        pltpu.sync_copy(erow_deg.at[s], didx)
        plsc.subcore_barrier()
        @pl.loop(0, DCHUNKS)
        def _(ch):
            pltpu.sync_copy(ones, degS.at[didx.at[ch]], add=True)

        plsc.subcore_barrier()
        row0 = c * HALF + s * ROWS_PT
        pltpu.sync_copy(degS.at[pl.ds(row0, ROWS_PT)], degL)

        @pl.loop(0, ROWS_PT // 16)
        def _(j):
            disL[pl.ds(j * 16, 16)] = _rsqrt16(degL[pl.ds(j * 16, 16)])

        pltpu.sync_copy(disL, disH.at[pl.ds(row0, ROWS_PT)])

    return setup1


@functools.lru_cache(maxsize=None)
def _make_setup2():
    @functools.partial(
        pl.kernel,
        out_type=(jax.ShapeDtypeStruct((NC * KCAP, D), f32),
                  jax.ShapeDtypeStruct((NPAD, D), f32)),
        mesh=_mesh(),
        scratch_types=[
            pltpu.VMEM((KCHUNKS, KCH), i32),
            pltpu.VMEM((KCHUNKS, KCH), i32),
            pltpu.VMEM((KCH, D), f32),
            pltpu.VMEM((64, D), f32),
            pltpu.VMEM((KCH + 16,), f32),
            pltpu.SemaphoreType.DMA,
            pltpu.SemaphoreType.DMA,
        ],
    )
    def setup2(x, disH, kidx32, kvx32, xk, v0, kidxL, kvxL, xbuf, zblock,
               dchunk, sem0, sem1):
        c = lax.axis_index("c")
        s = lax.axis_index("s")
        g = _wid()
        pltpu.sync_copy(kidx32.at[g], kidxL)
        pltpu.sync_copy(kvx32.at[g], kvxL)
        _zero_vmem(zblock, 64)
        row0 = c * HALF + s * ROWS_PT
        for b in range(ROWS_PT // 64):
            pltpu.sync_copy(zblock, v0.at[pl.ds(row0 + b * 64, 64)])
        plsc.subcore_barrier()
        for ch in range(KCHUNKS):
            pltpu.async_copy(x.at[kvxL.at[ch]], xbuf, sem0).wait()
            pltpu.async_copy(disH.at[kidxL.at[ch]],
                             dchunk.at[pl.ds(0, KCH)], sem1).wait()

            @pl.loop(0, KCH)
            def _(r):
                sc = dchunk[pl.ds(r, 16)][0]
                for j in range(D // 16):
                    xbuf[r, pl.ds(j * 16, 16)] = (
                        xbuf[r, pl.ds(j * 16, 16)] * sc)

            pltpu.sync_copy(xbuf, xk.at[pl.ds((g * KCHUNKS + ch) * KCH, KCH)])
            pltpu.sync_copy(xbuf, v0.at[kidxL.at[ch]])

    return setup2


@functools.lru_cache(maxsize=None)
def _make_iter(last):
    @functools.partial(
        pl.kernel,
        out_type=jax.ShapeDtypeStruct((NPAD, D), f32),
        mesh=_mesh(),
        scratch_types=[
            pltpu.VMEM_SHARED((HALF, D), f32),
            pltpu.VMEM((ECHUNKS, ECH), i32),
            pltpu.VMEM((ECHUNKS, ECH), i32),
            pltpu.VMEM((ECH, D), f32),
            pltpu.VMEM((ECH, D), f32),
            pltpu.VMEM((64, D), f32),
            pltpu.VMEM((KCHUNKS, KCH), i32),
            pltpu.VMEM((KCHUNKS, KCH), i32),
            pltpu.VMEM((ROWS_PT + 16,), f32),
            pltpu.SemaphoreType.DMA,
            pltpu.SemaphoreType.DMA,
        ],
    )
    def it(v_in, erow2d, ecol2d, disH, xk, kidx32, kvx32, x, v_out,
           accum, erowL, ecolL, gbuf0, gbuf1, zblock, kidxL, kvxL,
           scaleL, sem0, sem1):
        c = lax.axis_index("c")
        s = lax.axis_index("s")
        g = _wid()
        row0 = c * HALF + s * ROWS_PT
        pltpu.sync_copy(erow2d.at[g], erowL)
        pltpu.sync_copy(ecol2d.at[g], ecolL)
        pltpu.sync_copy(kidx32.at[g], kidxL)
        if last:
            pltpu.sync_copy(kvx32.at[g], kvxL)
        _zero_vmem(zblock, 64)
        for b in range(ROWS_PT // 64):
            pltpu.sync_copy(zblock,
                            accum.at[pl.ds(s * ROWS_PT + b * 64, 64)])
        pltpu.sync_copy(disH.at[pl.ds(row0, ROWS_PT)],
                        scaleL.at[pl.ds(0, ROWS_PT)])
        if not last:
            @pl.loop(0, ROWS_PT // 16)
            def _(j):
                dv = scaleL[pl.ds(j * 16, 16)]
                scaleL[pl.ds(j * 16, 16)] = dv * dv

        plsc.subcore_barrier()
        pltpu.async_copy(v_in.at[ecolL.at[0]], gbuf0, sem0)

        @pl.loop(0, ECHUNKS, step=2)
        def _(ch):
            pltpu.async_copy(v_in.at[ecolL.at[ch + 1]], gbuf1, sem1)
            pltpu.make_async_copy(v_in.at[ecolL.at[ch]], gbuf0, sem0).wait()
            pltpu.sync_copy(gbuf0, accum.at[erowL.at[ch]], add=True)

            @pl.when(ch + 2 < ECHUNKS)
            def _():
                pltpu.async_copy(v_in.at[ecolL.at[ch + 2]], gbuf0, sem0)

            pltpu.make_async_copy(v_in.at[ecolL.at[ch + 1]], gbuf1,
                                  sem1).wait()
            pltpu.sync_copy(gbuf1, accum.at[erowL.at[ch + 1]], add=True)

        plsc.subcore_barrier()
        for b in range(ROWS_PT // KCH):
            ob = gbuf0.at[pl.ds(0, KCH)]
            pltpu.sync_copy(accum.at[pl.ds(s * ROWS_PT + b * KCH, KCH)], ob)

            @pl.loop(0, KCH)
            def _(r):
                sc = scaleL[pl.ds(b * KCH + r, 16)][0]
                for j in range(D // 16):
                    gbuf0[r, pl.ds(j * 16, 16)] = (
                        gbuf0[r, pl.ds(j * 16, 16)] * sc)

            pltpu.sync_copy(ob, v_out.at[pl.ds(row0 + b * KCH, KCH)])
        plsc.subcore_barrier()
        for ch in range(KCHUNKS):
            kb = gbuf1.at[pl.ds(0, KCH)]
            if last:
                pltpu.async_copy(x.at[kvxL.at[ch]], kb, sem1).wait()
            else:
                pltpu.sync_copy(
                    xk.at[pl.ds((g * KCHUNKS + ch) * KCH, KCH)], kb)
            pltpu.sync_copy(kb, v_out.at[kidxL.at[ch]])

    return it


def _buf(r):
    return jnp.where(r < NREAL_HALF, r, r + ZPAD_W)


def _prep_inputs(x, edge_index, known_feature_mask):
    row = edge_index[0].astype(i32)
    col = edge_index[1].astype(i32)
    known = known_feature_mask.astype(i32)
    brow = _buf(row)
    bcol = _buf(col)
    bknown = _buf(known)

    npad_deg = EPAD_DEG - E
    pad_rows = ZPAD0 + (jnp.arange(npad_deg, dtype=i32) % ZPAD_W)
    erow_deg = jnp.concatenate([brow, pad_rows]).reshape(NS, DCHUNKS, ECH)

    side = (row >= NREAL_HALF).astype(i32)
    order = jnp.argsort(side, stable=True)
    n0 = E - jnp.sum(side)
    pos = jnp.arange(E, dtype=i32)
    pos_in_half = jnp.where(pos < n0, pos, pos - n0)
    half_id = (pos >= n0).astype(i32)
    lrow_s = (brow - side * HALF)[order]
    col_s = bcol[order]
    slot = jnp.arange(ECAP, dtype=i32)
    pad_lrow = slot % HALF
    pad_col = ZPAD0 + (slot % ZPAD_W)
    erow_cap = jnp.broadcast_to(pad_lrow, (NC, ECAP))
    ecol_cap = jnp.broadcast_to(pad_col, (NC, ECAP))
    erow_cap = erow_cap.at[half_id, pos_in_half].set(lrow_s)
    ecol_cap = ecol_cap.at[half_id, pos_in_half].set(col_s)
    erow2d = erow_cap.reshape(NW, ECHUNKS, ECH)
    ecol2d = ecol_cap.reshape(NW, ECHUNKS, ECH)

    kside = (known >= NREAL_HALF).astype(i32)
    korder = jnp.argsort(kside, stable=True)
    kn0 = K - jnp.sum(kside)
    kpos = jnp.arange(K, dtype=i32)
    kpos_in_half = jnp.where(kpos < kn0, kpos, kpos - kn0)
    khalf_id = (kpos >= kn0).astype(i32)
    kslot = jnp.arange(KCAP, dtype=i32)
    kpad = ZPAD1 + (kslot % ZPAD_W)
    kidx_cap = jnp.broadcast_to(kpad, (NC, KCAP))
    kidx_cap = kidx_cap.at[khalf_id, kpos_in_half].set(bknown[korder])
    kvx_cap = jnp.zeros((NC, KCAP), i32)
    kvx_cap = kvx_cap.at[khalf_id, kpos_in_half].set(known[korder])
    kidx32 = kidx_cap.reshape(NW, KCHUNKS, KCH)
    kvx32 = kvx_cap.reshape(NW, KCHUNKS, KCH)
    return erow_deg, erow2d, ecol2d, kidx32, kvx32


def kernel(x, edge_index, known_feature_mask):
    erow_deg, erow2d, ecol2d, kidx32, kvx32 = _prep_inputs(
        x, edge_index, known_feature_mask)
    disH = _make_setup1()(erow_deg)
    xkbuf, v = _make_setup2()(x, disH, kidx32, kvx32)
    step = _make_iter(False)
    for _ in range(ITERS - 1):
        v = step(v, erow2d, ecol2d, disH, xkbuf, kidx32, kvx32, x)
    out = _make_iter(True)(v, erow2d, ecol2d, disH, xkbuf, kidx32,
                           kvx32, x)
    return jnp.concatenate([out[:NREAL_HALF], out[HALF:HALF + NREAL_HALF]])
